# baseline (speedup 1.0000x reference)
import jax
import jax.numpy as jnp
from jax import lax
from jax.experimental import pallas as pl
from jax.experimental.pallas import tpu as pltpu
from jax.experimental.pallas import tpu_sc as plsc

C = 64
K = 512
HW = 512
NC = 2
NS = 16
NW = NC * NS
TPW = K // NW
SPS = HW // NS
JB = C // 16
LOSS_SCALE = 2.0 / (C * K * C)


def _vq_body(xt_hbm, xwb_hbm, etb_hbm, et64f_hbm, out_hbm, loss_hbm,
             xt_v, xw_v, et_v, et64f_v, out_v, stage_v, allst_v, s1s2_v,
             lstage_v, lall_v, lossv_v, sem0, sem1, sem2, sem3,
             stats_sh, loss_sh):
    core = lax.axis_index("c")
    s = lax.axis_index("s")
    w = core * NS + s
    zero = jnp.zeros((16,), jnp.float32)

    cp_xt = pltpu.make_async_copy(xt_hbm.at[pl.ds(s * SPS, SPS)], xt_v, sem0)
    cp_xt.start()
    cp_xw = pltpu.make_async_copy(xwb_hbm.at[w], xw_v, sem1)
    cp_xw.start()
    cp_et = pltpu.make_async_copy(etb_hbm.at[w], et_v, sem2)
    cp_et.start()
    cp_e64 = pltpu.make_async_copy(et64f_hbm, et64f_v, sem3)
    cp_e64.start()
    cp_xt.wait()

    s1p = [zero for _ in range(JB)]
    s2p = [zero for _ in range(JB)]
    for p in range(SPS):
        for jb in range(JB):
            v = xt_v[p, pl.ds(jb * 16, 16)]
            s1p[jb] = s1p[jb] + v
            s2p[jb] = s2p[jb] + v * v
    for jb in range(JB):
        stage_v[pl.ds(jb * 16, 16)] = s1p[jb]
        stage_v[pl.ds(64 + jb * 16, 16)] = s2p[jb]
    pltpu.sync_copy(stage_v, stats_sh.at[s])
    plsc.subcore_barrier()

    pltpu.sync_copy(stats_sh, allst_v)
    for r in range(2 * JB):
        acc = zero
        for srow in range(NS):
            acc = acc + allst_v[srow, pl.ds(r * 16, 16)]
        s1s2_v[pl.ds(r * 16, 16)] = acc

    cp_et.wait()
    minval = jnp.full((16,), 3.4e38, jnp.float32)
    minidx = jnp.zeros((16,), jnp.int32)
    for jb in range(JB):
        s1blk = s1s2_v[pl.ds(jb * 16, 16)]
        s2blk = s1s2_v[pl.ds(C + jb * 16, 16)]
        for jl in range(16):
            j = jb * 16 + jl
            s1j = s1blk[jl]
            s2j = s2blk[jl]
            e = et_v[j, :]
            d = (s2j - (2.0 * s1j) * e) + float(HW) * (e * e)
            m = d < minval
            minval = jnp.where(m, d, minval)
            minidx = jnp.where(m, jnp.full((16,), j, jnp.int32), minidx)

    cp_e64.wait()
    qs = zero
    q2 = zero
    for c in range(C):
        g = plsc.load_gather(et64f_v, [minidx + (c * C)])
        out_v[c, :] = g
        qs = qs + g
        q2 = q2 + g * g
    cp_xw.wait()
    xs = zero
    x2 = zero
    for j in range(C):
        v = xw_v[j, :]
        xs = xs + v
        x2 = x2 + v * v
    loss_acc = float(C) * q2 - (2.0 * qs) * xs + float(C) * x2

    pltpu.sync_copy(out_v, out_hbm.at[w])

    for pz in range(8):
        lstage_v[pl.ds(pz * 16, 16)] = zero
    lstage_v[pl.ds(0, 16)] = loss_acc
    pltpu.sync_copy(lstage_v, loss_sh.at[s])
    plsc.subcore_barrier()

    @pl.when(s == 0)
    def _final():
        pltpu.sync_copy(loss_sh, lall_v)
        acc = zero
        for r in range(NS):
            acc = acc + lall_v[r, pl.ds(0, 16)]
        total = plsc.cumsum(acc)[15] * LOSS_SCALE
        for pz in range(8):
            lossv_v[pl.ds(pz * 16, 16)] = zero
        lossv_v[pl.ds(0, 16)] = zero + total
        pltpu.sync_copy(lossv_v, loss_hbm.at[core])


_vq_call_cache = []


def _get_vq_call():
    if not _vq_call_cache:
        _vq_call_cache.append(_build_vq_call())
    return _vq_call_cache[0]


def _build_vq_call():
    return pl.kernel(
        _vq_body,
        out_type=(
            jax.ShapeDtypeStruct((NW, C, TPW), jnp.float32),
            jax.ShapeDtypeStruct((NC, 2 * C), jnp.float32),
        ),
        mesh=plsc.VectorSubcoreMesh(core_axis_name="c", subcore_axis_name="s",
                                    num_cores=NC, num_subcores=NS),
        compiler_params=pltpu.CompilerParams(needs_layout_passes=False),
        scratch_types=[
            pltpu.VMEM((SPS, C), jnp.float32),
            pltpu.VMEM((C, TPW), jnp.float32),
            pltpu.VMEM((C, TPW), jnp.float32),
            pltpu.VMEM((C * C,), jnp.float32),
            pltpu.VMEM((C, TPW), jnp.float32),
            pltpu.VMEM((2 * C,), jnp.float32),
            pltpu.VMEM((NS, 2 * C), jnp.float32),
            pltpu.VMEM((2 * C,), jnp.float32),
            pltpu.VMEM((2 * C,), jnp.float32),
            pltpu.VMEM((NS, 2 * C), jnp.float32),
            pltpu.VMEM((2 * C,), jnp.float32),
            pltpu.SemaphoreType.DMA,
            pltpu.SemaphoreType.DMA,
            pltpu.SemaphoreType.DMA,
            pltpu.SemaphoreType.DMA,
            pltpu.VMEM_SHARED((NS, 2 * C), jnp.float32),
            pltpu.VMEM_SHARED((NS, 2 * C), jnp.float32),
        ],
    )


def kernel(x, embeddings):
    b, c, h, w = x.shape
    x2d = x.reshape(c, h * w)
    xt = x2d.T
    xwb = x2d.reshape(c, NW, TPW).transpose(1, 0, 2)
    et = embeddings.T
    etb = et.reshape(c, NW, TPW).transpose(1, 0, 2)
    et64f = et[:, :C].reshape(-1)

    out_blk, loss_parts = _get_vq_call()(xt, xwb, etb, et64f)
    z_q = out_blk.transpose(1, 0, 2).reshape(b, c, h, w)
    return (z_q, loss_parts[0, 0] + loss_parts[1, 0])


# speedup vs baseline: 1.2251x; 1.2251x over previous
import jax
import jax.numpy as jnp
from jax import lax
from jax.experimental import pallas as pl
from jax.experimental.pallas import tpu as pltpu
from jax.experimental.pallas import tpu_sc as plsc

C = 64
K = 512
HW = 512
NS = 16
TPS = K // NS
NCH = TPS // 16
JB = C // 16
LOSS_SCALE = 2.0 / (C * K * C)


def _vq_body(xt_hbm, xwb_hbm, etb_hbm, et64f_hbm, out_hbm, loss_hbm,
             xt_v, xw_v, et_v, et64f_v, out_v, stage_v, allst_v, s1s2_v,
             lstage_v, lall_v, lossv_v, sem0, sem1, sem2, sem3,
             stats_sh, loss_sh):
    core = lax.axis_index("c")
    s = lax.axis_index("s")

    @pl.when(core == 0)
    def _run():
        zero = jnp.zeros((16,), jnp.float32)
        izero = jnp.zeros((16,), jnp.int32)
        iota = lax.iota(jnp.int32, 16)

        cp_xt = pltpu.make_async_copy(xt_hbm.at[pl.ds(s * TPS, TPS)], xt_v,
                                      sem0)
        cp_xt.start()
        cp_et = pltpu.make_async_copy(etb_hbm.at[s], et_v, sem1)
        cp_et.start()
        cp_e64 = pltpu.make_async_copy(et64f_hbm, et64f_v, sem2)
        cp_e64.start()
        cp_xw = pltpu.make_async_copy(xwb_hbm.at[s], xw_v, sem3)
        cp_xw.start()
        cp_xt.wait()

        def a_body(p, carry):
            pv = jnp.full((16,), p, jnp.int32)
            new = []
            for jb in range(JB):
                v = plsc.load_gather(xt_v, [pv, iota + (jb * 16)])
                new.append(carry[jb] + v)
            for jb in range(JB):
                v = plsc.load_gather(xt_v, [pv, iota + (jb * 16)])
                new.append(carry[JB + jb] + v * v)
            return tuple(new)

        stats = lax.fori_loop(0, TPS, a_body, (zero,) * (2 * JB))
        for jb in range(JB):
            stage_v[pl.ds(jb * 16, 16)] = stats[jb]
            stage_v[pl.ds(64 + jb * 16, 16)] = stats[JB + jb]
        pltpu.sync_copy(stage_v, stats_sh.at[s])
        plsc.subcore_barrier()

        pltpu.sync_copy(stats_sh, allst_v)

        def r_body(srow, carry):
            rv = jnp.full((16,), srow, jnp.int32)
            return tuple(
                carry[r] + plsc.load_gather(allst_v, [rv, iota + (r * 16)])
                for r in range(2 * JB))

        tot = lax.fori_loop(0, NS, r_body, (zero,) * (2 * JB))
        for r in range(2 * JB):
            s1s2_v[pl.ds(r * 16, 16)] = tot[r]

        cp_et.wait()

        def b_body(j, carry):
            jv = jnp.full((16,), j, jnp.int32)
            s1j = plsc.load_gather(s1s2_v, [jv])
            s2j = plsc.load_gather(s1s2_v, [jv + C])
            t1 = 2.0 * s1j
            out = []
            for t in range(NCH):
                mv, mi = carry[2 * t], carry[2 * t + 1]
                e = plsc.load_gather(et_v, [jv, iota + (t * 16)])
                d = (s2j - t1 * e) + float(HW) * (e * e)
                m = d < mv
                out.append(jnp.where(m, d, mv))
                out.append(jnp.where(m, jv, mi))
            return tuple(out)

        binit = (jnp.full((16,), 3.4e38, jnp.float32), izero) * NCH
        bres = lax.fori_loop(0, C, b_body, binit)
        minidx = [bres[1], bres[3]]

        cp_e64.wait()
        cp_xw.wait()

        def c_body(c, carry):
            cv = jnp.full((16,), c, jnp.int32)
            out = []
            for t in range(NCH):
                qs, q2, xs, x2 = carry[4 * t:4 * t + 4]
                g = plsc.load_gather(et64f_v, [minidx[t] + c * C])
                plsc.store_scatter(out_v, [cv, iota + (t * 16)], g)
                v = plsc.load_gather(xw_v, [cv, iota + (t * 16)])
                out.extend((qs + g, q2 + g * g, xs + v, x2 + v * v))
            return tuple(out)

        cres = lax.fori_loop(0, C, c_body, (zero,) * (4 * NCH))
        loss_acc = zero
        for t in range(NCH):
            qs, q2, xs, x2 = cres[4 * t:4 * t + 4]
            loss_acc = loss_acc + (float(C) * q2 - (2.0 * qs) * xs
                                   + float(C) * x2)

        pltpu.sync_copy(out_v, out_hbm.at[s])

        for pz in range(8):
            lstage_v[pl.ds(pz * 16, 16)] = zero
        lstage_v[pl.ds(0, 16)] = loss_acc
        pltpu.sync_copy(lstage_v, loss_sh.at[s])
        plsc.subcore_barrier()

        @pl.when(s == 0)
        def _final():
            pltpu.sync_copy(loss_sh, lall_v)

            def l_body(srow, acc):
                rv = jnp.full((16,), srow, jnp.int32)
                return acc + plsc.load_gather(lall_v, [rv, iota])

            acc = lax.fori_loop(0, NS, l_body, zero)
            total = plsc.cumsum(acc)[15] * LOSS_SCALE
            for pz in range(8):
                lossv_v[pl.ds(pz * 16, 16)] = zero
            lossv_v[pl.ds(0, 16)] = zero + total
            pltpu.sync_copy(lossv_v, loss_hbm)


_vq_call_cache = []


def _get_vq_call():
    if not _vq_call_cache:
        _vq_call_cache.append(_build_vq_call())
    return _vq_call_cache[0]


def _build_vq_call():
    return pl.kernel(
        _vq_body,
        out_type=(
            jax.ShapeDtypeStruct((NS, C, TPS), jnp.float32),
            jax.ShapeDtypeStruct((2 * C,), jnp.float32),
        ),
        mesh=plsc.VectorSubcoreMesh(core_axis_name="c", subcore_axis_name="s",
                                    num_cores=2, num_subcores=16),
        compiler_params=pltpu.CompilerParams(needs_layout_passes=False),
        scratch_types=[
            pltpu.VMEM((TPS, C), jnp.float32),
            pltpu.VMEM((C, TPS), jnp.float32),
            pltpu.VMEM((C, TPS), jnp.float32),
            pltpu.VMEM((C * C,), jnp.float32),
            pltpu.VMEM((C, TPS), jnp.float32),
            pltpu.VMEM((2 * C,), jnp.float32),
            pltpu.VMEM((NS, 2 * C), jnp.float32),
            pltpu.VMEM((2 * C,), jnp.float32),
            pltpu.VMEM((2 * C,), jnp.float32),
            pltpu.VMEM((NS, 2 * C), jnp.float32),
            pltpu.VMEM((2 * C,), jnp.float32),
            pltpu.SemaphoreType.DMA,
            pltpu.SemaphoreType.DMA,
            pltpu.SemaphoreType.DMA,
            pltpu.SemaphoreType.DMA,
            pltpu.VMEM_SHARED((NS, 2 * C), jnp.float32),
            pltpu.VMEM_SHARED((NS, 2 * C), jnp.float32),
        ],
    )


def kernel(x, embeddings):
    b, c, h, w = x.shape
    x2d = x.reshape(c, h * w)
    xt = x2d.T
    xwb = x2d.reshape(c, NS, TPS).transpose(1, 0, 2)
    et = embeddings.T
    etb = et.reshape(c, NS, TPS).transpose(1, 0, 2)
    et64f = et[:, :C].reshape(-1)

    out_blk, loss_vec = _get_vq_call()(xt, xwb, etb, et64f)
    z_q = out_blk.transpose(1, 0, 2).reshape(b, c, h, w)
    return (z_q, loss_vec[0])


# speedup vs baseline: 1.2294x; 1.0035x over previous
import jax
import jax.numpy as jnp
from jax import lax
from jax.experimental import pallas as pl
from jax.experimental.pallas import tpu as pltpu
from jax.experimental.pallas import tpu_sc as plsc

C = 64
K = 512
HW = 512
NS = 16
TPS = K // NS
NCH = TPS // 16
JB = C // 16
LOSS_SCALE = 2.0 / (C * K * C)


def _vq_body(xt_hbm, xwb_hbm, etb_hbm, et64f_hbm, out_hbm, loss_hbm,
             xt_v, xw_v, et_v, et64f_v, out_v, stage_v, allst_v, s1s2_v,
             lstage_v, lall_v, lossv_v, sem0, sem1, sem2, sem3,
             stats_sh, loss_sh):
    core = lax.axis_index("c")
    s = lax.axis_index("s")

    @pl.when(core == 0)
    def _run():
        zero = jnp.zeros((16,), jnp.float32)
        izero = jnp.zeros((16,), jnp.int32)
        iota = lax.iota(jnp.int32, 16)

        cp_xt = pltpu.make_async_copy(xt_hbm.at[pl.ds(s * TPS, TPS)], xt_v,
                                      sem0)
        cp_xt.start()
        cp_et = pltpu.make_async_copy(etb_hbm.at[s], et_v, sem1)
        cp_et.start()
        cp_e64 = pltpu.make_async_copy(et64f_hbm, et64f_v, sem2)
        cp_e64.start()
        cp_xw = pltpu.make_async_copy(xwb_hbm.at[s], xw_v, sem3)
        cp_xw.start()
        cp_xt.wait()

        def a_body(p, carry):
            pv = jnp.full((16,), p, jnp.int32)
            new = list(carry)
            for jb in range(JB):
                v = plsc.load_gather(xt_v, [pv, iota + (jb * 16)])
                new[jb] = new[jb] + v
                new[JB + jb] = new[JB + jb] + v * v
            return tuple(new)

        stats = lax.fori_loop(0, TPS, a_body, (zero,) * (2 * JB), unroll=4)
        for jb in range(JB):
            stage_v[pl.ds(jb * 16, 16)] = stats[jb]
            stage_v[pl.ds(64 + jb * 16, 16)] = stats[JB + jb]
        pltpu.sync_copy(stage_v, stats_sh.at[s])
        plsc.subcore_barrier()

        pltpu.sync_copy(stats_sh, allst_v)

        def r_body(srow, carry):
            rv = jnp.full((16,), srow, jnp.int32)
            return tuple(
                carry[r] + plsc.load_gather(allst_v, [rv, iota + (r * 16)])
                for r in range(2 * JB))

        tot = lax.fori_loop(0, NS, r_body, (zero,) * (2 * JB), unroll=4)
        for r in range(2 * JB):
            s1s2_v[pl.ds(r * 16, 16)] = tot[r]

        cp_et.wait()

        def b_body(j, carry):
            jv = jnp.full((16,), j, jnp.int32)
            s1j = plsc.load_gather(s1s2_v, [jv])
            s2j = plsc.load_gather(s1s2_v, [jv + C])
            t1 = 2.0 * s1j
            out = []
            for t in range(NCH):
                mv, mi = carry[2 * t], carry[2 * t + 1]
                e = plsc.load_gather(et_v, [jv, iota + (t * 16)])
                d = (s2j - t1 * e) + float(HW) * (e * e)
                m = d < mv
                out.append(jnp.where(m, d, mv))
                out.append(jnp.where(m, jv, mi))
            return tuple(out)

        binit = (jnp.full((16,), 3.4e38, jnp.float32), izero) * NCH
        bres = lax.fori_loop(0, C, b_body, binit, unroll=2)
        minidx = [bres[1], bres[3]]

        cp_e64.wait()
        cp_xw.wait()

        def c_body(c, carry):
            cv = jnp.full((16,), c, jnp.int32)
            out = []
            for t in range(NCH):
                qs, q2, xs, x2 = carry[4 * t:4 * t + 4]
                g = plsc.load_gather(et64f_v, [minidx[t] + c * C])
                plsc.store_scatter(out_v, [cv, iota + (t * 16)], g)
                v = plsc.load_gather(xw_v, [cv, iota + (t * 16)])
                out.extend((qs + g, q2 + g * g, xs + v, x2 + v * v))
            return tuple(out)

        cres = lax.fori_loop(0, C, c_body, (zero,) * (4 * NCH), unroll=2)
        loss_acc = zero
        for t in range(NCH):
            qs, q2, xs, x2 = cres[4 * t:4 * t + 4]
            loss_acc = loss_acc + (float(C) * q2 - (2.0 * qs) * xs
                                   + float(C) * x2)

        pltpu.sync_copy(out_v, out_hbm.at[s])

        for pz in range(8):
            lstage_v[pl.ds(pz * 16, 16)] = zero
        lstage_v[pl.ds(0, 16)] = loss_acc
        pltpu.sync_copy(lstage_v, loss_sh.at[s])
        plsc.subcore_barrier()

        @pl.when(s == 0)
        def _final():
            pltpu.sync_copy(loss_sh, lall_v)

            def l_body(srow, acc):
                rv = jnp.full((16,), srow, jnp.int32)
                return acc + plsc.load_gather(lall_v, [rv, iota])

            acc = lax.fori_loop(0, NS, l_body, zero, unroll=4)
            total = plsc.cumsum(acc)[15] * LOSS_SCALE
            for pz in range(8):
                lossv_v[pl.ds(pz * 16, 16)] = zero
            lossv_v[pl.ds(0, 16)] = zero + total
            pltpu.sync_copy(lossv_v, loss_hbm)


_vq_call_cache = []


def _get_vq_call():
    if not _vq_call_cache:
        _vq_call_cache.append(_build_vq_call())
    return _vq_call_cache[0]


def _build_vq_call():
    return pl.kernel(
        _vq_body,
        out_type=(
            jax.ShapeDtypeStruct((NS, C, TPS), jnp.float32),
            jax.ShapeDtypeStruct((2 * C,), jnp.float32),
        ),
        mesh=plsc.VectorSubcoreMesh(core_axis_name="c", subcore_axis_name="s",
                                    num_cores=2, num_subcores=16),
        compiler_params=pltpu.CompilerParams(needs_layout_passes=False),
        scratch_types=[
            pltpu.VMEM((TPS, C), jnp.float32),
            pltpu.VMEM((C, TPS), jnp.float32),
            pltpu.VMEM((C, TPS), jnp.float32),
            pltpu.VMEM((C * C,), jnp.float32),
            pltpu.VMEM((C, TPS), jnp.float32),
            pltpu.VMEM((2 * C,), jnp.float32),
            pltpu.VMEM((NS, 2 * C), jnp.float32),
            pltpu.VMEM((2 * C,), jnp.float32),
            pltpu.VMEM((2 * C,), jnp.float32),
            pltpu.VMEM((NS, 2 * C), jnp.float32),
            pltpu.VMEM((2 * C,), jnp.float32),
            pltpu.SemaphoreType.DMA,
            pltpu.SemaphoreType.DMA,
            pltpu.SemaphoreType.DMA,
            pltpu.SemaphoreType.DMA,
            pltpu.VMEM_SHARED((NS, 2 * C), jnp.float32),
            pltpu.VMEM_SHARED((NS, 2 * C), jnp.float32),
        ],
    )


def kernel(x, embeddings):
    b, c, h, w = x.shape
    x2d = x.reshape(c, h * w)
    xt = x2d.T
    xwb = x2d.reshape(c, NS, TPS).transpose(1, 0, 2)
    et = embeddings.T
    etb = et.reshape(c, NS, TPS).transpose(1, 0, 2)
    et64f = et[:, :C].reshape(-1)

    out_blk, loss_vec = _get_vq_call()(xt, xwb, etb, et64f)
    z_q = out_blk.transpose(1, 0, 2).reshape(b, c, h, w)
    return (z_q, loss_vec[0])
